# Initial kernel scaffold; baseline (speedup 1.0000x reference)
import jax
import jax.numpy as jnp
from jax.experimental import pallas as pl


def kernel(input, weight):
    raise NotImplementedError("write your pallas kernel here")


# speedup vs baseline: 1.3401x; 1.3401x over previous
import functools

import jax
import jax.numpy as jnp
from jax import lax
from jax.experimental import pallas as pl
from jax.experimental.pallas import tpu as pltpu
from jax.experimental.pallas import tpu_sc as plsc

NUM_EMB = 1_000_000
DIM = 16
BATCH = 16384
BAG = 50

NUM_CORES = 2
NUM_SUBCORES = 16
NW = NUM_CORES * NUM_SUBCORES
BPW = BATCH // NW
NBUF = 2


@functools.partial(
    pl.kernel,
    mesh=plsc.VectorSubcoreMesh(core_axis_name="c", subcore_axis_name="s"),
    out_type=jax.ShapeDtypeStruct((BATCH, DIM), jnp.float32),
    compiler_params=pltpu.CompilerParams(use_tc_tiling_on_sc=False),
    scratch_types=[
        pltpu.VMEM((BPW, BAG), jnp.int32),
        pltpu.VMEM((BPW, DIM), jnp.float32),
        pltpu.VMEM((BAG, DIM), jnp.float32),
        pltpu.VMEM((BAG, DIM), jnp.float32),
        pltpu.SemaphoreType.DMA,
        pltpu.SemaphoreType.DMA,
    ],
)
def _embedding_bag_sc(idx_hbm, tbl_hbm, out_hbm, idx_v, out_v,
                      rows0, rows1, sem0, sem1):
    wid = lax.axis_index("s") * NUM_CORES + lax.axis_index("c")
    base = wid * BPW

    pltpu.sync_copy(idx_hbm.at[pl.ds(base, BPW)], idx_v)

    rows = (rows0, rows1)
    sems = (sem0, sem1)

    def start(bag, b):
        pltpu.async_copy(tbl_hbm.at[idx_v.at[bag]], rows[b], sems[b])

    def finish(bag, b):
        pltpu.make_async_copy(tbl_hbm.at[idx_v.at[bag]], rows[b],
                              sems[b]).wait()
        r = rows[b]
        parts = []
        for c in range(5):
            acc = r[10 * c]
            for k in range(10 * c + 1, 10 * c + 10):
                acc = acc + r[k]
            parts.append(acc)
        total = (parts[0] + parts[1]) + (parts[2] + parts[3]) + parts[4]
        out_v[bag] = total * jnp.float32(1.0 / BAG)

    for b in range(NBUF):
        start(b, b)

    def body(i, carry):
        for b in range(NBUF):
            bag = NBUF * i + b
            finish(bag, b)
            start(bag + NBUF, b)
        return carry

    lax.fori_loop(0, BPW // NBUF - 1, body, 0)

    for b in range(NBUF):
        finish(BPW - NBUF + b, b)

    pltpu.sync_copy(out_v, out_hbm.at[pl.ds(base, BPW)])


def kernel(input, weight):
    idx = input.astype(jnp.int32)
    return _embedding_bag_sc(idx, weight)


# speedup vs baseline: 1.7055x; 1.2726x over previous
import functools

import jax
import jax.numpy as jnp
from jax import lax
from jax.experimental import pallas as pl
from jax.experimental.pallas import tpu as pltpu
from jax.experimental.pallas import tpu_sc as plsc

NUM_EMB = 1_000_000
DIM = 16
BATCH = 16384
BAG = 50

NUM_CORES = 2
NUM_SUBCORES = 16
NW = NUM_CORES * NUM_SUBCORES
BPW = BATCH // NW
PAIR = 2 * BAG
PPW = BPW // 2
NBUF = 8


@functools.partial(
    pl.kernel,
    mesh=plsc.VectorSubcoreMesh(core_axis_name="c", subcore_axis_name="s"),
    out_type=jax.ShapeDtypeStruct((BATCH, DIM), jnp.float32),
    compiler_params=pltpu.CompilerParams(use_tc_tiling_on_sc=False),
    scratch_types=[
        pltpu.VMEM((PPW, PAIR), jnp.int32),
        pltpu.VMEM((BPW, DIM), jnp.float32),
    ] + [pltpu.VMEM((PAIR, DIM), jnp.float32) for _ in range(NBUF)]
      + [pltpu.SemaphoreType.DMA for _ in range(NBUF)],
)
def _embedding_bag_sc(idx_hbm, tbl_hbm, out_hbm, idx_v, out_v, *bufs):
    rows = bufs[:NBUF]
    sems = bufs[NBUF:]
    wid = lax.axis_index("s") * NUM_CORES + lax.axis_index("c")

    pltpu.sync_copy(idx_hbm.at[pl.ds(wid * PPW, PPW)], idx_v)

    def start(p, b):
        pltpu.async_copy(tbl_hbm.at[idx_v.at[p]], rows[b], sems[b])

    def finish(p, b):
        pltpu.make_async_copy(tbl_hbm.at[idx_v.at[p]], rows[b],
                              sems[b]).wait()
        r = rows[b]
        for half in range(2):
            parts = []
            for c in range(5):
                base = 50 * half + 10 * c
                acc = r[base]
                for k in range(base + 1, base + 10):
                    acc = acc + r[k]
                parts.append(acc)
            total = (parts[0] + parts[1]) + (parts[2] + parts[3]) + parts[4]
            out_v[2 * p + half] = total * jnp.float32(1.0 / BAG)

    for b in range(NBUF):
        start(b, b)

    def body(i, carry):
        for b in range(NBUF):
            p = NBUF * i + b
            finish(p, b)
            start(p + NBUF, b)
        return carry

    lax.fori_loop(0, PPW // NBUF - 1, body, 0)

    for b in range(NBUF):
        finish(PPW - NBUF + b, b)

    pltpu.sync_copy(out_v, out_hbm.at[pl.ds(wid * BPW, BPW)])


def kernel(input, weight):
    idx = input.astype(jnp.int32).reshape(BATCH // 2, PAIR)
    return _embedding_bag_sc(idx, weight)
